# baseline (speedup 1.0000x reference)
import functools
import jax
import jax.numpy as jnp
from jax import lax
from jax.experimental import pallas as pl
from jax.experimental.pallas import tpu as pltpu
from jax.experimental.pallas import tpu_sc as plsc

D_MODEL = 64
SCALE = 8.0
LANES = 16
NUM_CORES = 2
NUM_SUBCORES = 16
NUM_WORKERS = NUM_CORES * NUM_SUBCORES
CHUNK = 128
NBUF = 8
ROW_UNROLL = 4


def _make_gather(batch: int):
    assert batch % (NUM_WORKERS * CHUNK) == 0
    b_per_w = batch // NUM_WORKERS
    n_chunks = b_per_w // CHUNK
    mesh = plsc.VectorSubcoreMesh(core_axis_name="c", subcore_axis_name="s")

    @functools.partial(
        pl.kernel,
        mesh=mesh,
        out_type=jax.ShapeDtypeStruct((batch, D_MODEL), jnp.float32),
        scratch_types=[
            pltpu.VMEM((n_chunks, CHUNK), jnp.int32),
            pltpu.VMEM((NBUF, CHUNK, D_MODEL), jnp.float32),
            pltpu.SemaphoreType.DMA((NBUF,)),
            pltpu.SemaphoreType.DMA((NBUF,)),
        ],
        compiler_params=pltpu.CompilerParams(use_tc_tiling_on_sc=False),
    )
    def gather_kernel(x_hbm, table_hbm, out_hbm, idx_v, rows_v, gsem, osem):
        wid = lax.axis_index("s") * NUM_CORES + lax.axis_index("c")
        base = wid * b_per_w
        pltpu.sync_copy(x_hbm.at[wid], idx_v)

        def fire_gather(f, bf):
            pltpu.async_copy(table_hbm.at[idx_v.at[f]], rows_v.at[bf],
                             gsem.at[bf])

        def wait_gather(b):
            pltpu.make_async_copy(table_hbm.at[idx_v.at[0]], rows_v.at[b],
                                  gsem.at[b]).wait()

        def fire_write(j, b):
            pltpu.async_copy(rows_v.at[b],
                             out_hbm.at[pl.ds(base + j * CHUNK, CHUNK)],
                             osem.at[b])

        def wait_write(b):
            pltpu.make_async_copy(rows_v.at[b], out_hbm.at[pl.ds(base, CHUNK)],
                                  osem.at[b]).wait()

        for i in range(NBUF - 2):
            fire_gather(i, i)

        def chunk_body(j, carry):
            b = lax.rem(j, NBUF)
            f = j + NBUF - 2
            bf = lax.rem(f, NBUF)

            @pl.when(f < n_chunks)
            def _():
                @pl.when(f >= NBUF)
                def _():
                    wait_write(bf)
                fire_gather(f, bf)

            wait_gather(b)

            def row_body(i, c):
                for r in range(ROW_UNROLL):
                    for d in range(D_MODEL // LANES):
                        sl = pl.ds(d * LANES, LANES)
                        rows_v[b, i * ROW_UNROLL + r, sl] = (
                            rows_v[b, i * ROW_UNROLL + r, sl] * SCALE)
                return c

            lax.fori_loop(0, CHUNK // ROW_UNROLL, row_body, 0)
            fire_write(j, b)
            return carry

        lax.fori_loop(0, n_chunks, chunk_body, 0)

        for c in range(n_chunks - NBUF, n_chunks):
            wait_write(c % NBUF)

    return gather_kernel


def kernel(x, table):
    batch, hist = x.shape
    total = batch * hist
    xf = x.reshape(NUM_WORKERS, total // (NUM_WORKERS * CHUNK), CHUNK)
    out = _make_gather(total)(xf, table)
    return out.reshape(batch, hist, D_MODEL)


# speedup vs baseline: 1.1032x; 1.1032x over previous
import functools
import jax
import jax.numpy as jnp
from jax import lax
from jax.experimental import pallas as pl
from jax.experimental.pallas import tpu as pltpu
from jax.experimental.pallas import tpu_sc as plsc

D_MODEL = 64
SCALE = 8.0
NUM_CORES = 2
NUM_SUBCORES = 16
NUM_WORKERS = NUM_CORES * NUM_SUBCORES
CHUNK = 128
NBUF = 8
OUT_W = 128


def _make_gather(batch: int):
    assert batch % (NUM_WORKERS * CHUNK) == 0
    b_per_w = batch // NUM_WORKERS
    n_chunks = b_per_w // CHUNK
    mesh = plsc.VectorSubcoreMesh(core_axis_name="c", subcore_axis_name="s")

    @functools.partial(
        pl.kernel,
        mesh=mesh,
        out_type=jax.ShapeDtypeStruct((batch, OUT_W), jnp.float32),
        scratch_types=[
            pltpu.VMEM((n_chunks, CHUNK), jnp.int32),
            pltpu.VMEM((NBUF, CHUNK, D_MODEL), jnp.float32),
            pltpu.SemaphoreType.DMA((NBUF,)),
            pltpu.SemaphoreType.DMA((NBUF,)),
        ],
        compiler_params=pltpu.CompilerParams(use_tc_tiling_on_sc=False),
    )
    def gather_kernel(x_hbm, table_hbm, out_hbm, idx_v, rows_v, gsem, osem):
        wid = lax.axis_index("s") * NUM_CORES + lax.axis_index("c")
        base = wid * b_per_w
        pltpu.sync_copy(x_hbm.at[wid], idx_v)

        def fire_gather(f, bf):
            pltpu.async_copy(table_hbm.at[idx_v.at[f]], rows_v.at[bf],
                             gsem.at[bf])

        def wait_gather(b):
            pltpu.make_async_copy(table_hbm.at[idx_v.at[0]], rows_v.at[b],
                                  gsem.at[b]).wait()

        def fire_write(j, b):
            pltpu.async_copy(
                rows_v.at[b],
                out_hbm.at[pl.ds(base + j * CHUNK, CHUNK), pl.ds(0, D_MODEL)],
                osem.at[b])

        def wait_write(b):
            pltpu.make_async_copy(
                rows_v.at[b],
                out_hbm.at[pl.ds(base, CHUNK), pl.ds(0, D_MODEL)],
                osem.at[b]).wait()

        for i in range(NBUF - 2):
            fire_gather(i, i)

        def chunk_body(j, carry):
            b = lax.rem(j, NBUF)
            f = j + NBUF - 2
            bf = lax.rem(f, NBUF)

            @pl.when(f < n_chunks)
            def _():
                @pl.when(f >= NBUF)
                def _():
                    wait_write(bf)
                fire_gather(f, bf)

            wait_gather(b)
            fire_write(j, b)
            return carry

        lax.fori_loop(0, n_chunks, chunk_body, 0)

        for c in range(n_chunks - NBUF, n_chunks):
            wait_write(c % NBUF)

    return gather_kernel


def _scale_relayout(lin, batch, hist):
    rows_per_blk = 8 * hist
    grid = batch // 8

    def body(in_ref, out_ref):
        out_ref[...] = in_ref[:, :D_MODEL].reshape(8, hist, D_MODEL) * SCALE

    return pl.pallas_call(
        body,
        grid=(grid,),
        in_specs=[pl.BlockSpec((rows_per_blk, OUT_W), lambda i: (i, 0))],
        out_specs=pl.BlockSpec((8, hist, D_MODEL), lambda i: (i, 0, 0)),
        out_shape=jax.ShapeDtypeStruct((batch, hist, D_MODEL), jnp.float32),
    )(lin)


def kernel(x, table):
    batch, hist = x.shape
    total = batch * hist
    xf = x.reshape(NUM_WORKERS, total // (NUM_WORKERS * CHUNK), CHUNK)
    lin = _make_gather(total)(xf, table)
    return _scale_relayout(lin, batch, hist)


# speedup vs baseline: 1.2135x; 1.0999x over previous
import functools
import jax
import jax.numpy as jnp
from jax import lax
from jax.experimental import pallas as pl
from jax.experimental.pallas import tpu as pltpu
from jax.experimental.pallas import tpu_sc as plsc

D_MODEL = 64
SCALE = 8.0
NUM_CORES = 2
NUM_SUBCORES = 16
NUM_WORKERS = NUM_CORES * NUM_SUBCORES
CHUNK = 128
NBUF = 8
OUT_W = 128


def _make_gather(batch: int):
    assert batch % (NUM_WORKERS * CHUNK) == 0
    b_per_w = batch // NUM_WORKERS
    n_chunks = b_per_w // CHUNK
    mesh = plsc.VectorSubcoreMesh(core_axis_name="c", subcore_axis_name="s")

    @functools.partial(
        pl.kernel,
        mesh=mesh,
        out_type=jax.ShapeDtypeStruct((batch, OUT_W), jnp.float32),
        scratch_types=[
            pltpu.VMEM((n_chunks, CHUNK), jnp.int32),
            pltpu.VMEM((NBUF, CHUNK, D_MODEL), jnp.float32),
            pltpu.SemaphoreType.DMA((NBUF,)),
            pltpu.SemaphoreType.DMA((NBUF,)),
        ],
        compiler_params=pltpu.CompilerParams(use_tc_tiling_on_sc=False),
    )
    def gather_kernel(x_hbm, table_hbm, out_hbm, idx_v, rows_v, gsem, osem):
        wid = lax.axis_index("s") * NUM_CORES + lax.axis_index("c")
        base = wid * b_per_w
        pltpu.sync_copy(x_hbm.at[wid], idx_v)

        def fire_gather(f, bf):
            pltpu.async_copy(table_hbm.at[idx_v.at[f]], rows_v.at[bf],
                             gsem.at[bf])

        def wait_gather(b):
            pltpu.make_async_copy(table_hbm.at[idx_v.at[0]], rows_v.at[b],
                                  gsem.at[b]).wait()

        def fire_write(j, b):
            pltpu.async_copy(
                rows_v.at[b],
                out_hbm.at[pl.ds(base + j * CHUNK, CHUNK), pl.ds(0, D_MODEL)],
                osem.at[b])

        def wait_write(b):
            pltpu.make_async_copy(
                rows_v.at[b],
                out_hbm.at[pl.ds(base, CHUNK), pl.ds(0, D_MODEL)],
                osem.at[b]).wait()

        for i in range(NBUF - 2):
            fire_gather(i, i)

        def chunk_body(j, carry):
            b = lax.rem(j, NBUF)
            f = j + NBUF - 2
            bf = lax.rem(f, NBUF)

            @pl.when(f < n_chunks)
            def _():
                @pl.when(f >= NBUF)
                def _():
                    wait_write(bf)
                fire_gather(f, bf)

            wait_gather(b)
            fire_write(j, b)
            return carry

        lax.fori_loop(0, n_chunks, chunk_body, 0)

        for c in range(n_chunks - NBUF, n_chunks):
            wait_write(c % NBUF)

    return gather_kernel


def _scale_relayout(lin3, batch, hist):
    blk = 16

    def body(in_ref, out_ref):
        out_ref[...] = in_ref[:, :, :D_MODEL] * SCALE

    return pl.pallas_call(
        body,
        grid=(batch // blk,),
        in_specs=[pl.BlockSpec((blk, hist, OUT_W), lambda i: (i, 0, 0))],
        out_specs=pl.BlockSpec((blk, hist, D_MODEL), lambda i: (i, 0, 0)),
        out_shape=jax.ShapeDtypeStruct((batch, hist, D_MODEL), jnp.float32),
    )(lin3)


def kernel(x, table):
    batch, hist = x.shape
    total = batch * hist
    xf = x.reshape(NUM_WORKERS, total // (NUM_WORKERS * CHUNK), CHUNK)
    lin = _make_gather(total)(xf, table)
    return _scale_relayout(lin.reshape(batch, hist, OUT_W), batch, hist)


# speedup vs baseline: 1.7445x; 1.4376x over previous
import functools
import jax
import jax.numpy as jnp
from jax import lax
from jax.experimental import pallas as pl
from jax.experimental.pallas import tpu as pltpu
from jax.experimental.pallas import tpu_sc as plsc

D_MODEL = 64
SCALE = 8.0
NUM_CORES = 2
NUM_SUBCORES = 16
NUM_WORKERS = NUM_CORES * NUM_SUBCORES
CHUNK = 128
NBUF = 8
OUT_W = 128


def _make_gather(batch: int):
    assert batch % (NUM_WORKERS * CHUNK) == 0
    b_per_w = batch // NUM_WORKERS
    n_chunks = b_per_w // CHUNK
    mesh = plsc.VectorSubcoreMesh(core_axis_name="c", subcore_axis_name="s")

    @functools.partial(
        pl.kernel,
        mesh=mesh,
        out_type=jax.ShapeDtypeStruct((batch, OUT_W), jnp.float32),
        scratch_types=[
            pltpu.VMEM((n_chunks, CHUNK), jnp.int32),
            pltpu.VMEM((NBUF, CHUNK, D_MODEL), jnp.float32),
            pltpu.SemaphoreType.DMA((NBUF,)),
            pltpu.SemaphoreType.DMA((NBUF,)),
        ],
        compiler_params=pltpu.CompilerParams(use_tc_tiling_on_sc=False),
    )
    def gather_kernel(x_hbm, table_hbm, out_hbm, idx_v, rows_v, gsem, osem):
        wid = lax.axis_index("s") * NUM_CORES + lax.axis_index("c")
        base = wid * b_per_w
        pltpu.sync_copy(x_hbm.at[wid], idx_v)

        def fire_gather(f, bf):
            pltpu.async_copy(table_hbm.at[idx_v.at[f]], rows_v.at[bf],
                             gsem.at[bf])

        def wait_gather(b):
            pltpu.make_async_copy(table_hbm.at[idx_v.at[0]], rows_v.at[b],
                                  gsem.at[b]).wait()

        def fire_write(j, b):
            pltpu.async_copy(
                rows_v.at[b],
                out_hbm.at[pl.ds(base + j * CHUNK, CHUNK), pl.ds(0, D_MODEL)],
                osem.at[b])

        def wait_write(b):
            pltpu.make_async_copy(
                rows_v.at[b],
                out_hbm.at[pl.ds(base, CHUNK), pl.ds(0, D_MODEL)],
                osem.at[b]).wait()

        for i in range(NBUF - 2):
            fire_gather(i, i)

        def chunk_body(j, carry):
            b = lax.rem(j, NBUF)
            f = j + NBUF - 2
            bf = lax.rem(f, NBUF)

            @pl.when(f < n_chunks)
            def _():
                @pl.when(f >= NBUF)
                def _():
                    wait_write(bf)
                fire_gather(f, bf)

            wait_gather(b)

            @plsc.parallel_loop(0, CHUNK, 1, unroll=8)
            def _(i):
                for d in range(D_MODEL // 16):
                    sl = pl.ds(d * 16, 16)
                    rows_v[b, i, sl] = rows_v[b, i, sl] * SCALE

            fire_write(j, b)
            return carry

        lax.fori_loop(0, n_chunks, chunk_body, 0)

        for c in range(n_chunks - NBUF, n_chunks):
            wait_write(c % NBUF)

    return gather_kernel


def kernel(x, table):
    batch, hist = x.shape
    total = batch * hist
    xf = x.reshape(NUM_WORKERS, total // (NUM_WORKERS * CHUNK), CHUNK)
    lin = _make_gather(total)(xf, table)
    return lin.reshape(batch, hist, OUT_W)[:, :, :D_MODEL]
